# baseline (speedup 1.0000x reference)
import functools

import jax
import jax.numpy as jnp
from jax import lax
from jax.experimental import pallas as pl
from jax.experimental.pallas import tpu as pltpu
from jax.experimental.pallas import tpu_sc as plsc

B, T, C = 128, 2048, 32
ALPHA = 2.0
LS = 0.05
TMAX = 3600.0

NC, NS = 2, 16
NW = NC * NS
B_PER_W = B // NW
CH = 256
CPB = T // CH
PPB = CPB // 2
NPAIRS = B_PER_W * PPB
STRIPS = CH // 16

_LN2 = 0.6931471805599453


def _log16(s):
    bits = lax.bitcast_convert_type(s, jnp.int32)
    e = ((bits >> 23) & 0xFF) - 127
    m = lax.bitcast_convert_type((bits & 0x007FFFFF) | 0x3F800000, jnp.float32)
    z = (m - 1.0) / (m + 1.0)
    z2 = z * z
    poly = 1.0 + z2 * ((1.0 / 3.0) + z2 * ((1.0 / 5.0) + z2 * ((1.0 / 7.0) + z2 * (1.0 / 9.0))))
    return e.astype(jnp.float32) * _LN2 + 2.0 * z * poly


@functools.lru_cache(maxsize=None)
def _build_sc_loss():
  @functools.partial(
    pl.kernel,
    out_type=(
        jax.ShapeDtypeStruct((NW * 16,), jnp.float32),
        jax.ShapeDtypeStruct((NW * 16,), jnp.float32),
    ),
    mesh=plsc.VectorSubcoreMesh(core_axis_name="c", subcore_axis_name="s",
                                num_cores=NC, num_subcores=NS),
    scratch_types=[
        pltpu.VMEM((2 * CH, C), jnp.float32),
        pltpu.VMEM((2 * CH,), jnp.float32),
        pltpu.VMEM((2 * CH,), jnp.float32),
        pltpu.VMEM((B,), jnp.int32),
        pltpu.VMEM((16,), jnp.float32),
        pltpu.VMEM((16,), jnp.float32),
        pltpu.SemaphoreType.DMA,
        pltpu.SemaphoreType.DMA,
    ],
    compiler_params=pltpu.CompilerParams(needs_layout_passes=False,
                                         use_tc_tiling_on_sc=True),
  )
  def _sc_loss(lg, lab, mk, tm, num_out, den_out,
               dbuf, tbuf, mbuf, lbuf, nbuf, dnbuf, sem0, sem1):
    wid = lax.axis_index("s") * NC + lax.axis_index("c")
    pltpu.sync_copy(lab, lbuf)

    iota = lax.iota(jnp.int32, 16)

    def copies(b, t0, slot, sem):
        return (
            pltpu.make_async_copy(lg.at[b, pl.ds(t0, CH)],
                                  dbuf.at[pl.ds(slot * CH, CH)], sem),
            pltpu.make_async_copy(tm.at[pl.ds(b * T + t0, CH)],
                                  tbuf.at[pl.ds(slot * CH, CH)], sem),
            pltpu.make_async_copy(mk.at[pl.ds(b * T + t0, CH)],
                                  mbuf.at[pl.ds(slot * CH, CH)], sem),
        )

    def start_chunk(b, t0, slot, sem):
        for cp in copies(b, t0, slot, sem):
            cp.start()

    def wait_chunk(b, t0, slot, sem):
        for cp in copies(b, t0, slot, sem):
            cp.wait()

    def compute_chunk(labv, slot, acc):
        base_row = slot * CH

        @plsc.parallel_loop(0, STRIPS, unroll=2, carry=acc)
        def body(i, acc):
            accn, accd = acc
            rows = (base_row + i * 16) + iota
            s = [jnp.zeros((16,), jnp.float32) for _ in range(4)]
            t = [jnp.zeros((16,), jnp.float32) for _ in range(4)]
            for c in range(C):
                rot = (iota + c) & (C - 1)
                v = plsc.load_gather(dbuf, [rows, rot])
                k = c & 3
                s[k] = s[k] + jnp.exp(v)
                t[k] = t[k] + v
            acc_s = (s[0] + s[1]) + (s[2] + s[3])
            acc_t = (t[0] + t[1]) + (t[2] + t[3])
            xl = plsc.load_gather(dbuf, [rows, labv])
            ce = _log16(acc_s) - (1.0 - LS) * xl - (LS / C) * acc_t
            roff = base_row + i * 16
            tv = plsc.load_gather(tbuf, [roff + iota])
            mv = plsc.load_gather(mbuf, [roff + iota])
            w = mv * ((1.0 + ALPHA) - (ALPHA / TMAX) * tv)
            return (accn + w * ce, accd + mv)

        return body

    b0 = wid * B_PER_W
    start_chunk(b0, 0, 0, sem0)

    def outer(k, acc):
        b = b0 + k // PPB
        t0 = (k % PPB) * (2 * CH)
        labv = plsc.load_gather(lbuf, [jnp.zeros((16,), jnp.int32) + b])
        start_chunk(b, t0 + CH, 1, sem1)
        wait_chunk(b, t0, 0, sem0)
        acc = compute_chunk(labv, 0, acc)

        @pl.when(k < NPAIRS - 1)
        def _():
            kn = k + 1
            start_chunk(b0 + kn // PPB, (kn % PPB) * (2 * CH), 0, sem0)

        wait_chunk(b, t0 + CH, 1, sem1)
        return compute_chunk(labv, 1, acc)

    acc = (jnp.zeros((16,), jnp.float32), jnp.zeros((16,), jnp.float32))
    acc = lax.fori_loop(0, NPAIRS, outer, acc)

    nbuf[...] = acc[0]
    dnbuf[...] = acc[1]
    pltpu.sync_copy(nbuf, num_out.at[pl.ds(wid * 16, 16)])
    pltpu.sync_copy(dnbuf, den_out.at[pl.ds(wid * 16, 16)])

  return _sc_loss


def kernel(step_logits, labels, bin_mask, bin_times):
    mk = bin_mask.astype(jnp.float32).reshape(B * T)
    tm = bin_times.reshape(B * T)
    nump, denp = _build_sc_loss()(step_logits, labels, mk, tm)
    num = jnp.sum(nump)
    den = jnp.sum(denp)
    return num / jnp.maximum(den, 1.0)


# speedup vs baseline: 1.4934x; 1.4934x over previous
import functools

import jax
import jax.numpy as jnp
from jax import lax
from jax.experimental import pallas as pl
from jax.experimental.pallas import tpu as pltpu
from jax.experimental.pallas import tpu_sc as plsc

B, T, C = 128, 2048, 32
ALPHA = 2.0
LS = 0.05
TMAX = 3600.0

NC, NS = 2, 16
NW = NC * NS
B_PER_W = B // NW
CH = 256
CPB = T // CH
PPB = CPB // 2
NPAIRS = B_PER_W * PPB
STRIPS = CH // 16

_LN2 = 0.6931471805599453


def _log16(s):
    bits = lax.bitcast_convert_type(s, jnp.int32)
    e = ((bits >> 23) & 0xFF) - 127
    m = lax.bitcast_convert_type((bits & 0x007FFFFF) | 0x3F800000, jnp.float32)
    z = (m - 1.0) / (m + 1.0)
    z2 = z * z
    poly = 1.0 + z2 * ((1.0 / 3.0) + z2 * ((1.0 / 5.0) + z2 * ((1.0 / 7.0) + z2 * (1.0 / 9.0))))
    return e.astype(jnp.float32) * _LN2 + 2.0 * z * poly


@functools.lru_cache(maxsize=None)
def _build_sc_loss():
  @functools.partial(
    pl.kernel,
    out_type=(
        jax.ShapeDtypeStruct((NW * 16,), jnp.float32),
        jax.ShapeDtypeStruct((NW * 16,), jnp.float32),
    ),
    mesh=plsc.VectorSubcoreMesh(core_axis_name="c", subcore_axis_name="s",
                                num_cores=NC, num_subcores=NS),
    scratch_types=[
        pltpu.VMEM((2 * CH, C), jnp.float32),
        pltpu.VMEM((2 * CH,), jnp.float32),
        pltpu.VMEM((2 * CH,), jnp.float32),
        pltpu.VMEM((B,), jnp.int32),
        pltpu.VMEM((16,), jnp.float32),
        pltpu.VMEM((16,), jnp.float32),
        pltpu.SemaphoreType.DMA,
        pltpu.SemaphoreType.DMA,
    ],
    compiler_params=pltpu.CompilerParams(needs_layout_passes=False,
                                         use_tc_tiling_on_sc=True),
  )
  def _sc_loss(lg, lab, mk, tm, num_out, den_out,
               dbuf, tbuf, mbuf, lbuf, nbuf, dnbuf, sem0, sem1):
    wid = lax.axis_index("s") * NC + lax.axis_index("c")
    pltpu.sync_copy(lab, lbuf)

    iota = lax.iota(jnp.int32, 16)

    def copies(b, t0, slot, sem):
        return (
            pltpu.make_async_copy(lg.at[pl.ds(b * T + t0, CH)],
                                  dbuf.at[pl.ds(slot * CH, CH)], sem),
            pltpu.make_async_copy(tm.at[pl.ds(b * T + t0, CH)],
                                  tbuf.at[pl.ds(slot * CH, CH)], sem),
            pltpu.make_async_copy(mk.at[pl.ds(b * T + t0, CH)],
                                  mbuf.at[pl.ds(slot * CH, CH)], sem),
        )

    def start_chunk(b, t0, slot, sem):
        for cp in copies(b, t0, slot, sem):
            cp.start()

    def wait_chunk(b, t0, slot, sem):
        for cp in copies(b, t0, slot, sem):
            cp.wait()

    def compute_chunk(labv, slot, acc):
        base_row = slot * CH

        @plsc.parallel_loop(0, STRIPS, unroll=4, carry=acc)
        def body(i, acc):
            accn, accd = acc
            rows = (base_row + i * 16) + iota
            s = [jnp.zeros((16,), jnp.float32) for _ in range(4)]
            t = [jnp.zeros((16,), jnp.float32) for _ in range(4)]
            for c in range(C):
                rot = (iota + c) & (C - 1)
                v = plsc.load_gather(dbuf, [rows, rot])
                k = c & 3
                s[k] = s[k] + jnp.exp(v)
                t[k] = t[k] + v
            acc_s = (s[0] + s[1]) + (s[2] + s[3])
            acc_t = (t[0] + t[1]) + (t[2] + t[3])
            xl = plsc.load_gather(dbuf, [rows, labv])
            ce = _log16(acc_s) - (1.0 - LS) * xl - (LS / C) * acc_t
            roff = base_row + i * 16
            tv = plsc.load_gather(tbuf, [roff + iota])
            mv = plsc.load_gather(mbuf, [roff + iota])
            w = mv * ((1.0 + ALPHA) - (ALPHA / TMAX) * tv)
            return (accn + w * ce, accd + mv)

        return body

    b0 = wid * B_PER_W
    start_chunk(b0, 0, 0, sem0)

    def outer(k, acc):
        b = b0 + k // PPB
        t0 = (k % PPB) * (2 * CH)
        labv = plsc.load_gather(lbuf, [jnp.zeros((16,), jnp.int32) + b])
        start_chunk(b, t0 + CH, 1, sem1)
        wait_chunk(b, t0, 0, sem0)
        acc = compute_chunk(labv, 0, acc)

        @pl.when(k < NPAIRS - 1)
        def _():
            kn = k + 1
            start_chunk(b0 + kn // PPB, (kn % PPB) * (2 * CH), 0, sem0)

        wait_chunk(b, t0 + CH, 1, sem1)
        return compute_chunk(labv, 1, acc)

    acc = (jnp.zeros((16,), jnp.float32), jnp.zeros((16,), jnp.float32))
    acc = lax.fori_loop(0, NPAIRS, outer, acc)

    nbuf[...] = acc[0]
    dnbuf[...] = acc[1]
    pltpu.sync_copy(nbuf, num_out.at[pl.ds(wid * 16, 16)])
    pltpu.sync_copy(dnbuf, den_out.at[pl.ds(wid * 16, 16)])

  return _sc_loss


def kernel(step_logits, labels, bin_mask, bin_times):
    lg = step_logits.reshape(B * T, C)
    mk = bin_mask.astype(jnp.float32).reshape(B * T)
    tm = bin_times.reshape(B * T)
    nump, denp = _build_sc_loss()(lg, labels, mk, tm)
    num = jnp.sum(nump)
    den = jnp.sum(denp)
    return num / jnp.maximum(den, 1.0)
